# baseline (speedup 1.0000x reference)
import functools

import jax
import jax.numpy as jnp
from jax import lax
from jax.experimental import pallas as pl
from jax.experimental.pallas import tpu as pltpu
from jax.experimental.pallas import tpu_sc as plsc

_D = 1024
_B = 4
_S = 2048
_EPS = 1e-6
_N = _B * _S
_NC = 2
_NS = 16
_NW = _NC * _NS
_TPW = _N // _NW
_G = 16
_NCHUNK = _TPW // _G
_L = 16
_DCH = _D // _L


def _xlane_sum(x):
    lanes = lax.iota(jnp.int32, _L)
    dnums = lax.GatherDimensionNumbers(
        offset_dims=(), collapsed_slice_dims=(0,), start_index_map=(0,))
    for k in (8, 4, 2, 1):
        x = x + lax.gather(x, (lanes ^ k)[:, None], dnums, slice_sizes=(1,),
                           mode=lax.GatherScatterMode.PROMISE_IN_BOUNDS)
    return x


def _tec_body(inp_hbm, table_hbm, pos_hbm, gamma_hbm, beta_hbm, out_hbm,
              idx_v, rows_v, pos_v, out_v, gamma_v, beta_v, sem):
    wid = lax.axis_index("s") * _NC + lax.axis_index("c")
    base = wid * _TPW
    sbase = lax.rem(base, _S)

    pltpu.sync_copy(inp_hbm.at[pl.ds(base, _TPW)], idx_v)
    pltpu.sync_copy(gamma_hbm, gamma_v)
    pltpu.sync_copy(beta_hbm, beta_v)

    def chunk(c, carry):
        pltpu.async_copy(table_hbm.at[idx_v.at[pl.ds(c * _G, _G)]],
                         rows_v, sem).wait()
        pltpu.sync_copy(pos_hbm.at[pl.ds(sbase + c * _G, _G)], pos_v)

        def row(r, rcarry):
            acc = jnp.zeros((_L,), jnp.float32)
            accq = jnp.zeros((_L,), jnp.float32)
            for j in range(_DCH):
                sl = pl.ds(j * _L, _L)
                x = rows_v[r, sl] + pos_v[r, sl]
                rows_v[r, sl] = x
                acc = acc + x
                accq = accq + x * x
            mv = _xlane_sum(acc) * (1.0 / _D)
            vv = _xlane_sum(accq) * (1.0 / _D) - mv * mv + _EPS
            yi = jnp.full((_L,), 0x5F3759DF, jnp.int32) - (
                plsc.bitcast(vv, jnp.int32) >> 1)
            y = plsc.bitcast(yi, jnp.float32)
            half_v = 0.5 * vv
            for _ in range(3):
                y = y * (1.5 - half_v * y * y)
            for j in range(_DCH):
                sl = pl.ds(j * _L, _L)
                out_v[r, sl] = (rows_v[r, sl] - mv) * y * gamma_v[sl] + beta_v[sl]
            return rcarry

        lax.fori_loop(0, _G, row, 0)
        pltpu.sync_copy(out_v, out_hbm.at[pl.ds(base + c * _G, _G)])
        return carry

    lax.fori_loop(0, _NCHUNK, chunk, 0)


@functools.partial(jax.jit, static_argnums=())
def kernel(input, mask, table, pos_embeds, gamma, beta):
    del mask
    inp_flat = input.reshape(_N).astype(jnp.int32)
    pos_flat = pos_embeds.reshape(_S, _D)
    mesh = plsc.VectorSubcoreMesh(core_axis_name="c", subcore_axis_name="s")
    run = pl.kernel(
        _tec_body,
        out_type=jax.ShapeDtypeStruct((_N, _D), jnp.float32),
        mesh=mesh,
        compiler_params=pltpu.CompilerParams(needs_layout_passes=False),
        scratch_types=[
            pltpu.VMEM((_TPW,), jnp.int32),
            pltpu.VMEM((_G, _D), jnp.float32),
            pltpu.VMEM((_G, _D), jnp.float32),
            pltpu.VMEM((_G, _D), jnp.float32),
            pltpu.VMEM((_D,), jnp.float32),
            pltpu.VMEM((_D,), jnp.float32),
            pltpu.SemaphoreType.DMA,
        ],
    )
    out = run(inp_flat, table, pos_flat, gamma, beta)
    return out.reshape(_B, _S, _D)


# speedup vs baseline: 1.1758x; 1.1758x over previous
import functools

import jax
import jax.numpy as jnp
from jax import lax
from jax.experimental import pallas as pl
from jax.experimental.pallas import tpu as pltpu
from jax.experimental.pallas import tpu_sc as plsc

_D = 1024
_B = 4
_S = 2048
_EPS = 1e-6
_NC = 2
_NS = 16
_NW = _NC * _NS
_SPW = _S // _NW
_SPC = 4
_G = _SPC * _B
_NCHUNK = _SPW // _SPC
_L = 16
_DCH = _D // _L


def _xlane_sum(x):
    lanes = lax.iota(jnp.int32, _L)
    dnums = lax.GatherDimensionNumbers(
        offset_dims=(), collapsed_slice_dims=(0,), start_index_map=(0,))
    for k in (8, 4, 2, 1):
        x = x + lax.gather(x, (lanes ^ k)[:, None], dnums, slice_sizes=(1,),
                           mode=lax.GatherScatterMode.PROMISE_IN_BOUNDS)
    return x


def _rsqrt(v):
    yi = jnp.full((_L,), 0x5F3759DF, jnp.int32) - (plsc.bitcast(v, jnp.int32) >> 1)
    y = plsc.bitcast(yi, jnp.float32)
    hv = 0.5 * v
    for _ in range(3):
        y = y * (1.5 - hv * y * y)
    return y


def _tec_body(inp_hbm, table_hbm, pos_hbm, gamma_hbm, beta_hbm, out_hbm,
              idx_v, idxg_v, rows_bufs, pos_bufs, out_bufs, gamma_v, beta_v,
              semg, semp, semo):
    wid = lax.axis_index("s") * _NC + lax.axis_index("c")
    sbase = wid * _SPW

    for b in range(_B):
        pltpu.sync_copy(inp_hbm.at[pl.ds(b * _S + sbase, _SPW)],
                        idx_v.at[pl.ds(b * _SPW, _SPW)])
    pltpu.sync_copy(gamma_hbm, gamma_v)
    pltpu.sync_copy(beta_hbm, beta_v)

    svec = lax.iota(jnp.int32, _L)
    for b in range(_B):
        for j in range(_SPW // _L):
            s = svec + (j * _L)
            dest = ((s >> 2) << 4) + (b * _SPC) + (s & 3)
            plsc.store_scatter(idxg_v, [dest],
                               idx_v[pl.ds(b * _SPW + j * _L, _L)])

    def issue(c, ph):
        pltpu.async_copy(
            table_hbm.at[idxg_v.at[pl.ds(c * _G, _G)]], rows_bufs[ph],
            semg[ph])
        pltpu.async_copy(
            pos_hbm.at[pl.ds(sbase + c * _SPC, _SPC)], pos_bufs[ph],
            semp[ph])

    issue(0, 0)

    def pair(i, carry):
        for ph in range(2):
            c = 2 * i + ph
            if ph == 0:
                issue(c + 1, 1)
            else:
                @pl.when(i < (_NCHUNK // 2 - 1))
                def _():
                    issue(c + 1, 0)
            rows_v = rows_bufs[ph]
            pos_v = pos_bufs[ph]
            out_v = out_bufs[ph]
            pltpu.make_async_copy(
                table_hbm.at[idxg_v.at[pl.ds(c * _G, _G)]], rows_v,
                semg[ph]).wait()
            pltpu.make_async_copy(
                pos_hbm.at[pl.ds(sbase + c * _SPC, _SPC)], pos_v,
                semp[ph]).wait()

            def p1(j, acc):
                accs, accqs = acc
                sl = pl.ds(j * _L, _L)
                pj = [pos_v[si, sl] for si in range(_SPC)]
                na, nq = [], []
                for r in range(_G):
                    x = rows_v[r, sl] + pj[r & 3]
                    rows_v[r, sl] = x
                    na.append(accs[r] + x)
                    nq.append(accqs[r] + x * x)
                return tuple(na), tuple(nq)

            zeros = tuple(jnp.zeros((_L,), jnp.float32) for _ in range(_G))
            accs, accqs = lax.fori_loop(0, _DCH, p1, (zeros, zeros))

            mvs, ys = [], []
            for r in range(_G):
                mv = _xlane_sum(accs[r]) * (1.0 / _D)
                vv = _xlane_sum(accqs[r]) * (1.0 / _D) - mv * mv + _EPS
                mvs.append(mv)
                ys.append(_rsqrt(vv))

            @pl.when(i >= 1)
            def _():
                pltpu.make_async_copy(
                    out_v, out_hbm.at[:, pl.ds(sbase, _SPC), :],
                    semo[ph]).wait()

            def p2(j, carry2):
                sl = pl.ds(j * _L, _L)
                g = gamma_v[sl]
                bt = beta_v[sl]
                for r in range(_G):
                    y = (rows_v[r, sl] - mvs[r]) * ys[r] * g + bt
                    out_v[r >> 2, r & 3, sl] = y
                return carry2

            lax.fori_loop(0, _DCH, p2, 0)
            pltpu.async_copy(
                out_v, out_hbm.at[:, pl.ds(sbase + c * _SPC, _SPC), :],
                semo[ph])
        return carry

    lax.fori_loop(0, _NCHUNK // 2, pair, 0)
    for ph in range(2):
        pltpu.make_async_copy(
            out_bufs[ph], out_hbm.at[:, pl.ds(sbase, _SPC), :],
            semo[ph]).wait()


@functools.partial(jax.jit, static_argnums=())
def kernel(input, mask, table, pos_embeds, gamma, beta):
    del mask
    inp = input.astype(jnp.int32).reshape(_B * _S)
    pos_flat = pos_embeds.reshape(_S, _D)
    mesh = plsc.VectorSubcoreMesh(core_axis_name="c", subcore_axis_name="s")
    run = pl.kernel(
        _tec_body,
        out_type=jax.ShapeDtypeStruct((_B, _S, _D), jnp.float32),
        mesh=mesh,
        compiler_params=pltpu.CompilerParams(needs_layout_passes=False),
        scratch_types=[
            pltpu.VMEM((_B * _SPW,), jnp.int32),
            pltpu.VMEM((_SPW * _B,), jnp.int32),
            [pltpu.VMEM((_G, _D), jnp.float32) for _ in range(2)],
            [pltpu.VMEM((_SPC, _D), jnp.float32) for _ in range(2)],
            [pltpu.VMEM((_B, _SPC, _D), jnp.float32) for _ in range(2)],
            pltpu.VMEM((_D,), jnp.float32),
            pltpu.VMEM((_D,), jnp.float32),
            [pltpu.SemaphoreType.DMA for _ in range(2)],
            [pltpu.SemaphoreType.DMA for _ in range(2)],
            [pltpu.SemaphoreType.DMA for _ in range(2)],
        ],
    )
    return run(inp, table, pos_flat, gamma, beta)


# speedup vs baseline: 3.5169x; 2.9910x over previous
import functools

import jax
import jax.numpy as jnp
from jax import lax
from jax.experimental import pallas as pl
from jax.experimental.pallas import tpu as pltpu
from jax.experimental.pallas import tpu_sc as plsc

_D = 1024
_B = 4
_S = 2048
_EPS = 1e-6
_NC = 2
_NS = 16
_NW = _NC * _NS
_SPW = _S // _NW
_SPC = 4
_G = _SPC * _B
_NCHUNK = _SPW // _SPC
_L = 16
_DCH = _D // _L


def _xlane_sum(x):
    lanes = lax.iota(jnp.int32, _L)
    dnums = lax.GatherDimensionNumbers(
        offset_dims=(), collapsed_slice_dims=(0,), start_index_map=(0,))
    for k in (8, 4, 2, 1):
        x = x + lax.gather(x, (lanes ^ k)[:, None], dnums, slice_sizes=(1,),
                           mode=lax.GatherScatterMode.PROMISE_IN_BOUNDS)
    return x


def _rsqrt(v):
    yi = jnp.full((_L,), 0x5F3759DF, jnp.int32) - (plsc.bitcast(v, jnp.int32) >> 1)
    y = plsc.bitcast(yi, jnp.float32)
    hv = 0.5 * v
    for _ in range(3):
        y = y * (1.5 - hv * y * y)
    return y


def _tec_body(inp_hbm, table_hbm, pos_hbm, gamma_hbm, beta_hbm, out_hbm,
              idx_v, idxg_v, rows_bufs, pos_bufs, out_bufs, gamma_v, beta_v,
              semg, semp, semo):
    wid = lax.axis_index("s") * _NC + lax.axis_index("c")
    sbase = wid * _SPW

    for b in range(_B):
        pltpu.sync_copy(inp_hbm.at[pl.ds(b * _S + sbase, _SPW)],
                        idx_v.at[pl.ds(b * _SPW, _SPW)])
    pltpu.sync_copy(gamma_hbm, gamma_v)
    pltpu.sync_copy(beta_hbm, beta_v)

    svec = lax.iota(jnp.int32, _L)
    for b in range(_B):
        for j in range(_SPW // _L):
            s = svec + (j * _L)
            dest = ((s >> 2) << 4) + (b * _SPC) + (s & 3)
            plsc.store_scatter(idxg_v, [dest],
                               idx_v[pl.ds(b * _SPW + j * _L, _L)])

    def issue(c, ph):
        pltpu.async_copy(
            table_hbm.at[idxg_v.at[pl.ds(c * _G, _G)]], rows_bufs[ph],
            semg[ph])
        pltpu.async_copy(
            pos_hbm.at[pl.ds(sbase + c * _SPC, _SPC)], pos_bufs[ph],
            semp[ph])

    issue(0, 0)

    def pair(i, carry):
        for ph in range(2):
            c = 2 * i + ph
            if ph == 0:
                issue(c + 1, 1)
            else:
                @pl.when(i < (_NCHUNK // 2 - 1))
                def _():
                    issue(c + 1, 0)
            rows_v = rows_bufs[ph]
            pos_v = pos_bufs[ph]
            out_v = out_bufs[ph]
            pltpu.make_async_copy(
                table_hbm.at[idxg_v.at[pl.ds(c * _G, _G)]], rows_v,
                semg[ph]).wait()
            pltpu.make_async_copy(
                pos_hbm.at[pl.ds(sbase + c * _SPC, _SPC)], pos_v,
                semp[ph]).wait()

            def p1(j, acc):
                accs, accqs = acc
                sl = pl.ds(j * _L, _L)
                pj = [pos_v[si, sl] for si in range(_SPC)]
                na, nq = list(accs), list(accqs)
                for h in range(2):
                    xs = [rows_v[8 * h + t, sl] + pj[(8 * h + t) & 3]
                          for t in range(8)]
                    for t in range(8):
                        r = 8 * h + t
                        rows_v[r, sl] = xs[t]
                        na[r] = na[r] + xs[t]
                        nq[r] = nq[r] + xs[t] * xs[t]
                return tuple(na), tuple(nq)

            zeros = tuple(jnp.zeros((_L,), jnp.float32) for _ in range(_G))
            accs, accqs = plsc.parallel_loop(
                0, _DCH, carry=(zeros, zeros))(p1)

            mvs, ys = [], []
            for r in range(_G):
                mv = _xlane_sum(accs[r]) * (1.0 / _D)
                vv = _xlane_sum(accqs[r]) * (1.0 / _D) - mv * mv + _EPS
                mvs.append(mv)
                ys.append(_rsqrt(vv))

            @pl.when(i >= 1)
            def _():
                pltpu.make_async_copy(
                    out_v, out_hbm.at[:, pl.ds(sbase, _SPC), :],
                    semo[ph]).wait()

            def p2(j):
                sl = pl.ds(j * _L, _L)
                g = gamma_v[sl]
                bt = beta_v[sl]
                for h in range(2):
                    xs = [rows_v[8 * h + t, sl] for t in range(8)]
                    vs = [(xs[t] - mvs[8 * h + t]) * ys[8 * h + t] * g + bt
                          for t in range(8)]
                    for t in range(8):
                        r = 8 * h + t
                        out_v[r >> 2, r & 3, sl] = vs[t]

            plsc.parallel_loop(0, _DCH)(p2)
            pltpu.async_copy(
                out_v, out_hbm.at[:, pl.ds(sbase + c * _SPC, _SPC), :],
                semo[ph])
        return carry

    lax.fori_loop(0, _NCHUNK // 2, pair, 0)
    for ph in range(2):
        pltpu.make_async_copy(
            out_bufs[ph], out_hbm.at[:, pl.ds(sbase, _SPC), :],
            semo[ph]).wait()


@functools.partial(jax.jit, static_argnums=())
def kernel(input, mask, table, pos_embeds, gamma, beta):
    del mask
    inp = input.astype(jnp.int32).reshape(_B * _S)
    pos_flat = pos_embeds.reshape(_S, _D)
    mesh = plsc.VectorSubcoreMesh(core_axis_name="c", subcore_axis_name="s")
    run = pl.kernel(
        _tec_body,
        out_type=jax.ShapeDtypeStruct((_B, _S, _D), jnp.float32),
        mesh=mesh,
        compiler_params=pltpu.CompilerParams(needs_layout_passes=False),
        scratch_types=[
            pltpu.VMEM((_B * _SPW,), jnp.int32),
            pltpu.VMEM((_SPW * _B,), jnp.int32),
            [pltpu.VMEM((_G, _D), jnp.float32) for _ in range(2)],
            [pltpu.VMEM((_SPC, _D), jnp.float32) for _ in range(2)],
            [pltpu.VMEM((_B, _SPC, _D), jnp.float32) for _ in range(2)],
            pltpu.VMEM((_D,), jnp.float32),
            pltpu.VMEM((_D,), jnp.float32),
            [pltpu.SemaphoreType.DMA for _ in range(2)],
            [pltpu.SemaphoreType.DMA for _ in range(2)],
            [pltpu.SemaphoreType.DMA for _ in range(2)],
        ],
    )
    return run(inp, table, pos_flat, gamma, beta)
